# baseline (speedup 1.0000x reference)
import functools

import jax
import jax.numpy as jnp
import numpy as np
from jax import lax
from jax.experimental import pallas as pl
from jax.experimental.pallas import tpu as pltpu
from jax.experimental.pallas import tpu_sc as plsc

NC = 2
NS = 16
L = 16
NBUF = 3


def kernel(tokens, actions, tok_embed0, tok_embed1, tok_embed2,
           action_embed, level_embed, pos_embed):
  B, T, _ = tokens.shape
  D = tok_embed0.shape[1]
  V = tok_embed0.shape[0]
  NA = action_embed.shape[0]
  NW = NC * NS
  TPB = T // NW
  R = 4 * TPB
  G = 2
  NB = B // G
  GT = G * TPB * 3
  GA = G * TPB
  NV = D // L

  colperm = np.array([b * 32 + (q % 2) * 16 + q // 2
                      for b in range(D // 32) for q in range(32)])
  table = jnp.concatenate(
      [tok_embed0, tok_embed1, tok_embed2],
      axis=0).astype(jnp.bfloat16)[:, colperm]
  table = lax.bitcast_convert_type(
      table.reshape(3 * V, D // 2, 2), jnp.int32)

  idx = jnp.stack(
      [tokens[:, :, 0], tokens[:, :, 1] + V, tokens[:, :, 2] + 2 * V],
      axis=0)
  idx = (idx.reshape(3, NB, G, NW, TPB)
         .transpose(3, 1, 2, 4, 0)
         .reshape(NW, NB * GT))
  aidx = (actions.reshape(NB, G, NW, TPB)
          .transpose(2, 0, 1, 3)
          .reshape(NW, NB * GA))

  mesh = plsc.VectorSubcoreMesh(
      core_axis_name="c", subcore_axis_name="s", num_cores=NC,
      num_subcores=NS)

  @functools.partial(
      pl.kernel,
      out_type=jax.ShapeDtypeStruct((B, 4 * T, D), jnp.float32),
      mesh=mesh,
      compiler_params=pltpu.CompilerParams(needs_layout_passes=False),
      scratch_types=[
          pltpu.VMEM((NB * GT,), jnp.int32),
          pltpu.VMEM((NB * GA,), jnp.int32),
          pltpu.VMEM((NBUF, GT, D // 2), jnp.int32),
          pltpu.VMEM((NBUF, G * R, D), jnp.float32),
          pltpu.VMEM((NA, D), jnp.float32),
          pltpu.VMEM((R, D), jnp.float32),
          pltpu.SemaphoreType.DMA((NBUF,)),
          pltpu.SemaphoreType.DMA((NBUF,)),
      ],
  )
  def k(tab_hbm, idx_hbm, aidx_hbm, act_hbm, lvl_hbm, pos_hbm, out_hbm,
        idx_v, aidx_v, rows32, outbuf, act_v, posbuf, gsems, wsems):
    wid = lax.axis_index("s") * NC + lax.axis_index("c")
    t0 = wid * TPB

    pltpu.sync_copy(idx_hbm.at[wid], idx_v)
    pltpu.sync_copy(aidx_hbm.at[wid], aidx_v)
    pltpu.sync_copy(act_hbm, act_v)
    pltpu.sync_copy(pos_hbm.at[pl.ds(4 * t0, R)], posbuf)
    pltpu.sync_copy(lvl_hbm, outbuf.at[0, pl.ds(0, 4)])

    @plsc.parallel_loop(0, R, unroll=4)
    def poslvl(r):
      j = lax.rem(r, 4)
      for v in range(NV):
        sl = pl.ds(L * v, L)
        posbuf[r, sl] = posbuf[r, sl] + outbuf[0, j, sl]

    laneseq = lax.iota(jnp.int32, 16)

    def start_gather(gi, buf):
      pltpu.async_copy(tab_hbm.at[idx_v.at[pl.ds(gi * GT, GT)]],
                       rows32.at[buf], gsems.at[buf])

    def wait_gather(buf):
      pltpu.make_async_copy(tab_hbm.at[idx_v.at[pl.ds(0, GT)]],
                            rows32.at[buf], gsems.at[buf]).wait()

    def start_writes(gi, buf):
      for g in range(G):
        pltpu.async_copy(outbuf.at[buf, pl.ds(g * R, R)],
                         out_hbm.at[gi * G + g, pl.ds(4 * t0, R)],
                         wsems.at[buf])

    def wait_writes(buf):
      for g in range(G):
        pltpu.make_async_copy(outbuf.at[buf, pl.ds(g * R, R)],
                              out_hbm.at[0, pl.ds(0, R)],
                              wsems.at[buf]).wait()

    for i in range(NBUF - 1):
      start_gather(i, i)

    @pl.loop(0, NB)
    def group(gi):
      buf = lax.rem(gi, NBUF)
      wait_gather(buf)

      @pl.when(gi >= NBUF)
      def drain_out():
        wait_writes(buf)

      @plsc.parallel_loop(0, TPB, unroll=2)
      def per_ti(ti):
        for g in range(G):
          for j in range(3):
            srow = (g * TPB + ti) * 3 + j
            orow = g * R + 4 * ti + j
            r = 4 * ti + j
            for v2 in range(D // 32):
              x = rows32[buf, srow, pl.ds(L * v2, L)]
              lo = lax.bitcast_convert_type(
                  lax.shift_left(x, 16), jnp.float32)
              hi = lax.bitcast_convert_type(
                  lax.bitwise_and(x, jnp.int32(-65536)), jnp.float32)
              sl_lo = pl.ds(32 * v2, L)
              sl_hi = pl.ds(32 * v2 + L, L)
              outbuf[buf, orow, sl_lo] = lo + posbuf[r, sl_lo]
              outbuf[buf, orow, sl_hi] = hi + posbuf[r, sl_hi]

      @plsc.parallel_loop(0, GA, unroll=2)
      def per_act(k):
        g = lax.div(k, TPB)
        ti = lax.rem(k, TPB)
        a_spl = plsc.load_gather(
            aidx_v, [jnp.full((16,), gi * GA, jnp.int32) + k])
        orow = g * R + 4 * ti + 3
        r = 4 * ti + 3
        for v in range(NV):
          sl = pl.ds(L * v, L)
          val = plsc.load_gather(act_v, [a_spl, laneseq + L * v])
          outbuf[buf, orow, sl] = val + posbuf[r, sl]

      start_writes(gi, buf)
      nbuf = lax.rem(gi + NBUF - 1, NBUF)

      @pl.when(gi + NBUF - 1 < NB)
      def prefetch():
        start_gather(gi + NBUF - 1, nbuf)

    for b in range(NBUF):
      wait_writes(b)

  return k(table, idx, aidx, action_embed, level_embed, pos_embed)


# speedup vs baseline: 1.2643x; 1.2643x over previous
import functools

import jax
import jax.numpy as jnp
import numpy as np
from jax import lax
from jax.experimental import pallas as pl
from jax.experimental.pallas import tpu as pltpu
from jax.experimental.pallas import tpu_sc as plsc

NC = 2
NS = 16
L = 16
NBUF = 3


def kernel(tokens, actions, tok_embed0, tok_embed1, tok_embed2,
           action_embed, level_embed, pos_embed):
  B, T, _ = tokens.shape
  D = tok_embed0.shape[1]
  V = tok_embed0.shape[0]
  NW = NC * NS
  TPB = T // NW
  R = 4 * TPB
  G = 2
  NB = B // G
  GI = G * R
  NV = D // L

  colperm = np.array([b * 32 + (q % 2) * 16 + q // 2
                      for b in range(D // 32) for q in range(32)])
  table = jnp.concatenate(
      [tok_embed0, tok_embed1, tok_embed2, action_embed],
      axis=0).astype(jnp.bfloat16)[:, colperm]
  table = lax.bitcast_convert_type(
      table.reshape(3 * V + action_embed.shape[0], D // 2, 2), jnp.int32)
  idx = jnp.stack(
      [tokens[:, :, 0], tokens[:, :, 1] + V, tokens[:, :, 2] + 2 * V,
       actions + 3 * V], axis=0)
  idx = (idx.reshape(4, NB, G, NW, TPB)
         .transpose(3, 1, 2, 4, 0)
         .reshape(NW, NB * GI))

  mesh = plsc.VectorSubcoreMesh(
      core_axis_name="c", subcore_axis_name="s", num_cores=NC,
      num_subcores=NS)

  @functools.partial(
      pl.kernel,
      out_type=jax.ShapeDtypeStruct((B, 4 * T, D), jnp.float32),
      mesh=mesh,
      scratch_types=[
          pltpu.VMEM((NB * GI,), jnp.int32),
          pltpu.VMEM((NBUF, GI, D // 2), jnp.int32),
          pltpu.VMEM((NBUF, GI, D), jnp.float32),
          pltpu.VMEM((R, D), jnp.float32),
          pltpu.VMEM((4, D), jnp.float32),
          pltpu.SemaphoreType.DMA((NBUF,)),
          pltpu.SemaphoreType.DMA((NBUF,)),
      ],
  )
  def k(tab_hbm, idx_hbm, lvl_hbm, pos_hbm, out_hbm,
        idx_v, rows32, outbuf, posbuf, lvlbuf, gsems, wsems):
    wid = lax.axis_index("s") * NC + lax.axis_index("c")
    t0 = wid * TPB

    pltpu.sync_copy(idx_hbm.at[wid], idx_v)
    pltpu.sync_copy(pos_hbm.at[pl.ds(4 * t0, R)], posbuf)
    pltpu.sync_copy(lvl_hbm, lvlbuf)

    @plsc.parallel_loop(0, R, unroll=4)
    def poslvl(r):
      j = lax.rem(r, 4)
      for v in range(NV):
        sl = pl.ds(L * v, L)
        posbuf[r, sl] = posbuf[r, sl] + lvlbuf[j, sl]

    def start_gather(gi, buf):
      pltpu.async_copy(tab_hbm.at[idx_v.at[pl.ds(gi * GI, GI)]],
                       rows32.at[buf], gsems.at[buf])

    def wait_gather(buf):
      pltpu.make_async_copy(tab_hbm.at[idx_v.at[pl.ds(0, GI)]],
                            rows32.at[buf], gsems.at[buf]).wait()

    def start_writes(gi, buf):
      for g in range(G):
        pltpu.async_copy(outbuf.at[buf, pl.ds(g * R, R)],
                         out_hbm.at[gi * G + g, pl.ds(4 * t0, R)],
                         wsems.at[buf])

    def wait_writes(buf):
      for g in range(G):
        pltpu.make_async_copy(outbuf.at[buf, pl.ds(g * R, R)],
                              out_hbm.at[0, pl.ds(0, R)],
                              wsems.at[buf]).wait()

    for i in range(NBUF - 1):
      start_gather(i, i)

    @pl.loop(0, NB)
    def group(gi):
      buf = lax.rem(gi, NBUF)
      wait_gather(buf)

      @pl.when(gi >= NBUF)
      def drain_out():
        wait_writes(buf)

      @plsc.parallel_loop(0, R, unroll=4)
      def per_row(r):
        for g in range(G):
          row = g * R + r
          for v2 in range(D // 32):
            x = rows32[buf, row, pl.ds(L * v2, L)]
            lo = lax.bitcast_convert_type(lax.shift_left(x, 16), jnp.float32)
            hi = lax.bitcast_convert_type(
                lax.bitwise_and(x, jnp.int32(-65536)), jnp.float32)
            sl_lo = pl.ds(32 * v2, L)
            sl_hi = pl.ds(32 * v2 + L, L)
            outbuf[buf, row, sl_lo] = lo + posbuf[r, sl_lo]
            outbuf[buf, row, sl_hi] = hi + posbuf[r, sl_hi]

      start_writes(gi, buf)
      nbuf = lax.rem(gi + NBUF - 1, NBUF)

      @pl.when(gi + NBUF - 1 < NB)
      def prefetch():
        start_gather(gi + NBUF - 1, nbuf)

    for b in range(NBUF):
      wait_writes(b)

  return k(table, idx, level_embed, pos_embed)


# speedup vs baseline: 2.2377x; 1.7699x over previous
import functools

import jax
import jax.numpy as jnp
import numpy as np
from jax import lax
from jax.experimental import pallas as pl
from jax.experimental.pallas import tpu as pltpu
from jax.experimental.pallas import tpu_sc as plsc

NC = 2
NS = 16
L = 16
NBUF = 3


def kernel(tokens, actions, tok_embed0, tok_embed1, tok_embed2,
           action_embed, level_embed, pos_embed):
  B, T, _ = tokens.shape
  D = tok_embed0.shape[1]
  V = tok_embed0.shape[0]
  NW = NC * NS
  TPB = T // NW
  R = 4 * TPB
  G = 2
  NB = B // G
  GI = G * R
  NV = D // L

  colperm = np.array([b * 32 + (q % 2) * 16 + q // 2
                      for b in range(D // 32) for q in range(32)])
  table = jnp.concatenate(
      [tok_embed0, tok_embed1, tok_embed2, action_embed],
      axis=0).astype(jnp.bfloat16)[:, colperm]
  table = lax.bitcast_convert_type(
      table.reshape(3 * V + action_embed.shape[0], D // 2, 2), jnp.int32)
  idx = jnp.stack(
      [tokens[:, :, 0], tokens[:, :, 1] + V, tokens[:, :, 2] + 2 * V,
       actions + 3 * V], axis=0)
  idx = (idx.reshape(4, NB, G, NW, TPB)
         .transpose(3, 1, 2, 4, 0)
         .reshape(NW, NB * GI))

  mesh = plsc.VectorSubcoreMesh(
      core_axis_name="c", subcore_axis_name="s", num_cores=NC,
      num_subcores=NS)

  @functools.partial(
      pl.kernel,
      out_type=jax.ShapeDtypeStruct((B, 4 * T, D), jnp.float32),
      mesh=mesh,
      scratch_types=[
          pltpu.VMEM((NB * GI,), jnp.int32),
          pltpu.VMEM((NBUF, GI, D // 2), jnp.int32),
          pltpu.VMEM((NBUF, GI, D), jnp.float32),
          pltpu.VMEM((R, D), jnp.float32),
          pltpu.VMEM((4, D), jnp.float32),
          pltpu.SemaphoreType.DMA((NBUF,)),
          pltpu.SemaphoreType.DMA((NBUF,)),
      ],
  )
  def k(tab_hbm, idx_hbm, lvl_hbm, pos_hbm, out_hbm,
        idx_v, rows32, outbuf, posbuf, lvlbuf, gsems, wsems):
    wid = lax.axis_index("s") * NC + lax.axis_index("c")
    t0 = wid * TPB

    pltpu.sync_copy(idx_hbm.at[wid], idx_v)
    pltpu.sync_copy(pos_hbm.at[pl.ds(4 * t0, R)], posbuf)
    pltpu.sync_copy(lvl_hbm, lvlbuf)

    @plsc.parallel_loop(0, R, unroll=4)
    def poslvl(r):
      j = lax.rem(r, 4)
      for v in range(NV):
        sl = pl.ds(L * v, L)
        posbuf[r, sl] = posbuf[r, sl] + lvlbuf[j, sl]

    def start_gather(gi, buf):
      pltpu.async_copy(tab_hbm.at[idx_v.at[pl.ds(gi * GI, GI)]],
                       rows32.at[buf], gsems.at[buf])

    def wait_gather(buf):
      pltpu.make_async_copy(tab_hbm.at[idx_v.at[pl.ds(0, GI)]],
                            rows32.at[buf], gsems.at[buf]).wait()

    def start_writes(gi, buf):
      for g in range(G):
        pltpu.async_copy(outbuf.at[buf, pl.ds(g * R, R)],
                         out_hbm.at[gi * G + g, pl.ds(4 * t0, R)],
                         wsems.at[buf])

    def wait_writes(buf):
      for g in range(G):
        pltpu.make_async_copy(outbuf.at[buf, pl.ds(g * R, R)],
                              out_hbm.at[0, pl.ds(0, R)],
                              wsems.at[buf]).wait()

    for i in range(NBUF - 1):
      start_gather(i, i)

    @pl.loop(0, NB)
    def group(gi):
      buf = lax.rem(gi, NBUF)
      wait_gather(buf)

      @pl.when(gi >= NBUF)
      def drain_out():
        wait_writes(buf)

      @plsc.parallel_loop(0, R, unroll=2)
      def per_row(r):
        for g in range(G):
          row = g * R + r
          for v2 in range(D // 32):
            x = rows32[buf, row, pl.ds(L * v2, L)]
            lo = lax.bitcast_convert_type(lax.shift_left(x, 16), jnp.float32)
            hi = lax.bitcast_convert_type(
                lax.bitwise_and(x, jnp.int32(-65536)), jnp.float32)
            sl_lo = pl.ds(32 * v2, L)
            sl_hi = pl.ds(32 * v2 + L, L)
            outbuf[buf, row, sl_lo] = lo + posbuf[r, sl_lo]
            outbuf[buf, row, sl_hi] = hi + posbuf[r, sl_hi]

      start_writes(gi, buf)
      nbuf = lax.rem(gi + NBUF - 1, NBUF)

      @pl.when(gi + NBUF - 1 < NB)
      def prefetch():
        start_gather(gi + NBUF - 1, nbuf)

    for b in range(NBUF):
      wait_writes(b)

  return k(table, idx, level_embed, pos_embed)
